# baseline (speedup 1.0000x reference)
import functools

import jax
import jax.numpy as jnp
from jax import lax
from jax.experimental import pallas as pl
from jax.experimental.pallas import tpu as pltpu
from jax.experimental.pallas import tpu_sc as plsc

B = 16
NQ = 20000
G = 4
E = 17
K = 4
F = G * K

NC, NS, L = 2, 16, 16
NW = NC * NS

NCHUNK = 1
BC = B // NCHUNK

Q = 384
TPB = 19968 // Q
CP = BC * TPB
PLO, PXTRA = divmod(CP, NW)
QT = TPB * Q
QTW = NQ - QT

_sc_mesh = plsc.VectorSubcoreMesh(
    core_axis_name="c", subcore_axis_name="s", num_cores=NC, num_subcores=NS
)


_mx, _mn = jnp.maximum, jnp.minimum


def _sort4(e0, e1, e2, e3):
  a = _mx(e0, e1); b = _mn(e0, e1)
  c = _mx(e2, e3); d = _mn(e2, e3)
  m1 = _mx(a, c); t = _mn(a, c)
  u = _mx(b, d); m4 = _mn(b, d)
  return m1, _mx(t, u), _mn(t, u), m4


def _merge_top4(a, b):
  a1, a2, a3, a4 = a
  b1, b2, b3, b4 = b
  p1 = _mx(a1, b4); p2 = _mx(a2, b3); p3 = _mx(a3, b2); p4 = _mx(a4, b1)
  q1 = _mx(p1, p3); q3 = _mn(p1, p3)
  q2 = _mx(p2, p4); q4 = _mn(p2, p4)
  return _mx(q1, q2), _mn(q1, q2), _mx(q3, q4), _mn(q3, q4)


def _insert1(m, v):
  m1, m2, m3, m4 = m
  o1 = _mx(m1, v); c = _mn(m1, v)
  o2 = _mx(m2, c); c = _mn(m2, c)
  o3 = _mx(m3, c); c = _mn(m3, c)
  return o1, o2, o3, _mx(m4, c)


def _top4_of_17(col):
  ms = []
  for j in range(E):
    v = col(j)
    out = []
    cur = v
    full = len(ms) == K
    for i, t in enumerate(ms):
      out.append(_mx(t, cur))
      if not (full and i == len(ms) - 1):
        cur = _mn(t, cur)
    if not full:
      out.append(cur)
    ms = out[:K]
  return ms


def _make_sc_topk(b0):
  @functools.partial(
      pl.kernel,
      out_type=jax.ShapeDtypeStruct((BC, F, NQ), jnp.float32),
      mesh=_sc_mesh,
      scratch_types=[
          pltpu.VMEM((E, G, Q), jnp.float32),
          pltpu.VMEM((E, G, Q), jnp.float32),
          pltpu.VMEM((F, Q), jnp.float32),
          pltpu.VMEM((F, Q), jnp.float32),
          pltpu.VMEM((E, G, QTW), jnp.float32),
          pltpu.VMEM((F, QTW), jnp.float32),
          pltpu.SemaphoreType.DMA,
          pltpu.SemaphoreType.DMA,
          pltpu.SemaphoreType.DMA,
          pltpu.SemaphoreType.DMA,
      ],
      compiler_params=pltpu.CompilerParams(needs_layout_passes=False),
  )
  def _sc_topk(dist_hbm, stat_hbm, buf0, buf1, obuf0, obuf1, tbuf, toutbuf,
               sem0, sem1, osem0, osem1):
    w = lax.axis_index("s") * NC + lax.axis_index("c")
    p0g = w * PLO + jnp.minimum(w, PXTRA)
    nps = PLO + jnp.where(w < PXTRA, 1, 0)

    def src(p):
      b = p // TPB
      t = p - b * TPB
      return dist_hbm.at[b0 + b, :, :, pl.ds(t * Q, Q)]

    def topk_sweep(buf, obuf, ngroups):
      @pl.loop(0, ngroups, unroll=2)
      def _g(qi):
        a0 = qi * L
        for g in range(G):
          ms = _top4_of_17(lambda e: buf[e, g, pl.ds(a0, L)])
          for k in range(K):
            obuf[g * K + k, pl.ds(a0, L)] = ms[k]

    def dst(p):
      b = p // TPB
      t = p - b * TPB
      return stat_hbm.at[b, :, pl.ds(t * Q, Q)]

    def step(p, buf, obuf, sem, osem):
      pg = p0g + p
      pltpu.make_async_copy(src(pg), buf, sem).wait()

      @pl.when(p >= 2)
      def _():
        pltpu.make_async_copy(obuf, dst(pg), osem).wait()

      topk_sweep(buf, obuf, Q // L)

      @pl.when(p + 2 < nps)
      def _():
        pltpu.async_copy(src(pg + 2), buf, sem)

      pltpu.async_copy(obuf, dst(pg), osem)

    pltpu.async_copy(src(p0g), buf0, sem0)
    pltpu.async_copy(src(p0g + 1), buf1, sem1)

    @pl.loop(0, nps)
    def _body(p):
      @pl.when(p % 2 == 0)
      def _even():
        step(p, buf0, obuf0, sem0, osem0)

      @pl.when(p % 2 == 1)
      def _odd():
        step(p, buf1, obuf1, sem1, osem1)

    pltpu.make_async_copy(obuf0, dst(p0g), osem0).wait()
    pltpu.make_async_copy(obuf1, dst(p0g), osem1).wait()

    @pl.when(w < BC)
    def _tail():
      pltpu.sync_copy(dist_hbm.at[b0 + w, :, :, pl.ds(QT, QTW)], tbuf)
      topk_sweep(tbuf, toutbuf, QTW // L)
      pltpu.sync_copy(toutbuf, stat_hbm.at[w, :, pl.ds(QT, QTW)])

  return _sc_topk


_sc_chunks = [_make_sc_topk(c * BC) for c in range(NCHUNK)]


def _tc_body(stat_ref, w1_ref, b1_ref, w2_ref, b2_ref, out_ref):
  x = stat_ref[0]
  h = lax.dot_general(
      w1_ref[...], x, (((1,), (0,)), ((), ())),
      preferred_element_type=jnp.float32,
  )
  h = jnp.maximum(h + b1_ref[...], 0.0)
  y = lax.dot_general(
      w2_ref[...], h, (((0,), (0,)), ((), ())),
      preferred_element_type=jnp.float32,
  ) + b2_ref[...]
  out_ref[...] = jax.nn.sigmoid(y)[None]


_tc_mlp = pl.pallas_call(
    _tc_body,
    grid=(BC,),
    in_specs=[
        pl.BlockSpec((1, F, NQ), lambda i: (i, 0, 0)),
        pl.BlockSpec((64, F), lambda i: (0, 0)),
        pl.BlockSpec((64, 1), lambda i: (0, 0)),
        pl.BlockSpec((64, 1), lambda i: (0, 0)),
        pl.BlockSpec((1, 1), lambda i: (0, 0)),
    ],
    out_specs=pl.BlockSpec((1, 1, NQ), lambda i: (i, 0, 0)),
    out_shape=jax.ShapeDtypeStruct((BC, 1, NQ), jnp.float32),
)


def kernel(dist_softmax, W1, b1, W2, b2):
  dist_t = jnp.transpose(dist_softmax, (0, 3, 2, 1))
  w1r = W1.reshape(64, G, K + 1)
  w1e = (w1r[:, :, :K] + 0.25 * w1r[:, :, K:]).reshape(64, F)
  b1c = b1.reshape(64, 1)
  w2c = W2.reshape(64, 1)
  b2c = b2.reshape(1, 1)
  outs = []
  for c in range(NCHUNK):
    stat3 = _sc_chunks[c](dist_t)
    outs.append(_tc_mlp(stat3, w1e, b1c, w2c, b2c))
  return jnp.concatenate(outs, axis=0).reshape(B, NQ)


# speedup vs baseline: 1.0827x; 1.0827x over previous
import functools

import jax
import jax.numpy as jnp
from jax import lax
from jax.experimental import pallas as pl
from jax.experimental.pallas import tpu as pltpu
from jax.experimental.pallas import tpu_sc as plsc

B = 16
NQ = 20000
G = 4
E = 17
K = 4
F = G * K

NC, NS, L = 2, 16, 16
NW = NC * NS

NCHUNK = 1
BC = B // NCHUNK

Q = 384
TPB = 19968 // Q
CP = BC * TPB
PLO, PXTRA = divmod(CP, NW)
QT = TPB * Q
QTW = NQ - QT

_sc_mesh = plsc.VectorSubcoreMesh(
    core_axis_name="c", subcore_axis_name="s", num_cores=NC, num_subcores=NS
)


_mx, _mn = jnp.maximum, jnp.minimum


def _sort4(e0, e1, e2, e3):
  a = _mx(e0, e1); b = _mn(e0, e1)
  c = _mx(e2, e3); d = _mn(e2, e3)
  m1 = _mx(a, c); t = _mn(a, c)
  u = _mx(b, d); m4 = _mn(b, d)
  return m1, _mx(t, u), _mn(t, u), m4


def _merge_top4(a, b):
  a1, a2, a3, a4 = a
  b1, b2, b3, b4 = b
  p1 = _mx(a1, b4); p2 = _mx(a2, b3); p3 = _mx(a3, b2); p4 = _mx(a4, b1)
  q1 = _mx(p1, p3); q3 = _mn(p1, p3)
  q2 = _mx(p2, p4); q4 = _mn(p2, p4)
  return _mx(q1, q2), _mn(q1, q2), _mx(q3, q4), _mn(q3, q4)


def _insert1(m, v):
  m1, m2, m3, m4 = m
  o1 = _mx(m1, v); c = _mn(m1, v)
  o2 = _mx(m2, c); c = _mn(m2, c)
  o3 = _mx(m3, c); c = _mn(m3, c)
  return o1, o2, o3, _mx(m4, c)


def _top4_of_17(col):
  ms = []
  for j in range(E):
    v = col(j)
    out = []
    cur = v
    full = len(ms) == K
    for i, t in enumerate(ms):
      out.append(_mx(t, cur))
      if not (full and i == len(ms) - 1):
        cur = _mn(t, cur)
    if not full:
      out.append(cur)
    ms = out[:K]
  return ms


def _make_sc_topk(b0):
  @functools.partial(
      pl.kernel,
      out_type=jax.ShapeDtypeStruct((BC, F, NQ), jnp.float32),
      mesh=_sc_mesh,
      scratch_types=[
          pltpu.VMEM((E, G, Q), jnp.float32),
          pltpu.VMEM((E, G, Q), jnp.float32),
          pltpu.VMEM((F, Q), jnp.float32),
          pltpu.VMEM((F, Q), jnp.float32),
          pltpu.VMEM((E, G, QTW), jnp.float32),
          pltpu.VMEM((F, QTW), jnp.float32),
          pltpu.SemaphoreType.DMA,
          pltpu.SemaphoreType.DMA,
          pltpu.SemaphoreType.DMA,
          pltpu.SemaphoreType.DMA,
      ],
      compiler_params=pltpu.CompilerParams(needs_layout_passes=False),
  )
  def _sc_topk(dist_hbm, stat_hbm, buf0, buf1, obuf0, obuf1, tbuf, toutbuf,
               sem0, sem1, osem0, osem1):
    w = lax.axis_index("s") * NC + lax.axis_index("c")
    p0g = w * PLO + jnp.minimum(w, PXTRA)
    nps = PLO + jnp.where(w < PXTRA, 1, 0)

    def src(p):
      b = p // TPB
      t = p - b * TPB
      return dist_hbm.at[b0 + b, :, :, pl.ds(t * Q, Q)]

    def topk_sweep(buf, obuf, ngroups):
      @pl.loop(0, ngroups)
      def _g(qi):
        a0 = qi * L
        for g in range(G):
          ms = _top4_of_17(lambda e: buf[e, g, pl.ds(a0, L)])
          for k in range(K):
            obuf[g * K + k, pl.ds(a0, L)] = ms[k]

    def dst(p):
      b = p // TPB
      t = p - b * TPB
      return stat_hbm.at[b, :, pl.ds(t * Q, Q)]

    def step(p, buf, obuf, sem, osem):
      pg = p0g + p
      pltpu.make_async_copy(src(pg), buf, sem).wait()

      @pl.when(p >= 2)
      def _():
        pltpu.make_async_copy(obuf, dst(pg), osem).wait()

      topk_sweep(buf, obuf, Q // L)

      @pl.when(p + 2 < nps)
      def _():
        pltpu.async_copy(src(pg + 2), buf, sem)

      pltpu.async_copy(obuf, dst(pg), osem)

    pltpu.async_copy(src(p0g), buf0, sem0)
    pltpu.async_copy(src(p0g + 1), buf1, sem1)

    @pl.when(w < BC)
    def _tail():
      pltpu.sync_copy(dist_hbm.at[b0 + w, :, :, pl.ds(QT, QTW)], tbuf)
      topk_sweep(tbuf, toutbuf, QTW // L)
      pltpu.sync_copy(toutbuf, stat_hbm.at[w, :, pl.ds(QT, QTW)])

    @pl.loop(0, nps)
    def _body(p):
      @pl.when(p % 2 == 0)
      def _even():
        step(p, buf0, obuf0, sem0, osem0)

      @pl.when(p % 2 == 1)
      def _odd():
        step(p, buf1, obuf1, sem1, osem1)

    pltpu.make_async_copy(obuf0, dst(p0g), osem0).wait()
    pltpu.make_async_copy(obuf1, dst(p0g), osem1).wait()

  return _sc_topk


_sc_chunks = [_make_sc_topk(c * BC) for c in range(NCHUNK)]


def _tc_body(stat_ref, w1_ref, b1_ref, w2_ref, b2_ref, out_ref):
  x = stat_ref[0]
  h = lax.dot_general(
      w1_ref[...], x, (((1,), (0,)), ((), ())),
      preferred_element_type=jnp.float32,
  )
  h = jnp.maximum(h + b1_ref[...], 0.0)
  y = lax.dot_general(
      w2_ref[...], h, (((0,), (0,)), ((), ())),
      preferred_element_type=jnp.float32,
  ) + b2_ref[...]
  out_ref[...] = jax.nn.sigmoid(y)[None]


_tc_mlp = pl.pallas_call(
    _tc_body,
    grid=(BC,),
    in_specs=[
        pl.BlockSpec((1, F, NQ), lambda i: (i, 0, 0)),
        pl.BlockSpec((64, F), lambda i: (0, 0)),
        pl.BlockSpec((64, 1), lambda i: (0, 0)),
        pl.BlockSpec((64, 1), lambda i: (0, 0)),
        pl.BlockSpec((1, 1), lambda i: (0, 0)),
    ],
    out_specs=pl.BlockSpec((1, 1, NQ), lambda i: (i, 0, 0)),
    out_shape=jax.ShapeDtypeStruct((BC, 1, NQ), jnp.float32),
)


def kernel(dist_softmax, W1, b1, W2, b2):
  dist_t = jnp.transpose(dist_softmax, (0, 3, 2, 1))
  w1r = W1.reshape(64, G, K + 1)
  w1e = (w1r[:, :, :K] + 0.25 * w1r[:, :, K:]).reshape(64, F)
  b1c = b1.reshape(64, 1)
  w2c = W2.reshape(64, 1)
  b2c = b2.reshape(1, 1)
  outs = []
  for c in range(NCHUNK):
    stat3 = _sc_chunks[c](dist_t)
    outs.append(_tc_mlp(stat3, w1e, b1c, w2c, b2c))
  return jnp.concatenate(outs, axis=0).reshape(B, NQ)
